# Initial kernel scaffold; baseline (speedup 1.0000x reference)
import jax
import jax.numpy as jnp
from jax.experimental import pallas as pl


def kernel(points, W_rel, b_rel, W_dist, b_dist, W_dens, b_dens, W_out, b_out):
    raise NotImplementedError("write your pallas kernel here")


# speedup vs baseline: 39.8091x; 39.8091x over previous
import jax
import jax.numpy as jnp
from jax import lax
from jax.experimental import pallas as pl

EMBED_DIM = 384
SUB = EMBED_DIM // 3
B, N = 16, 2048
ROWS = 256
T = N // ROWS


def _body(pts_ref, ptst_ref, mrel_ref, vdist_ref, vdens_ref, cvec_ref, out_ref):
    t = pl.program_id(1)
    pts = pts_ref[0]
    rows = pts_ref[0, pl.ds(t * ROWS, ROWS), :]
    ptst = ptst_ref[0]

    cen = jnp.mean(pts, axis=0, keepdims=True)
    x2r = jnp.sum(rows * rows, axis=1, keepdims=True)
    x2c = jnp.sum(ptst * ptst, axis=0, keepdims=True)
    dot = jnp.dot(rows, ptst, preferred_element_type=jnp.float32)
    d2 = jnp.maximum(x2r + x2c - 2.0 * dot, 0.0)

    ri = t * ROWS + lax.broadcasted_iota(jnp.int32, (ROWS, N), 0)
    ci = lax.broadcasted_iota(jnp.int32, (ROWS, N), 1)
    D = jnp.where(ri == ci, jnp.inf, d2)

    ssum = jnp.zeros((ROWS, 1), jnp.float32)
    for k in range(3):
        m = jnp.min(D, axis=1, keepdims=True)
        ssum = ssum + jnp.sqrt(m)
        if k < 2:
            sel = jnp.where(D == m, ci, jnp.int32(2**30))
            cmin = jnp.min(sel, axis=1, keepdims=True)
            D = jnp.where(ci == cmin, jnp.inf, D)
    dens = ssum * (1.0 / 3.0)

    rel = rows - cen
    cd = jnp.sqrt(jnp.sum(rel * rel, axis=1, keepdims=True))

    acc = cvec_ref[...] + cd * vdist_ref[...] + dens * vdens_ref[...]
    acc = acc + rel[:, 0:1] * mrel_ref[0:1, :]
    acc = acc + rel[:, 1:2] * mrel_ref[1:2, :]
    acc = acc + rel[:, 2:3] * mrel_ref[2:3, :]
    out_ref[0] = acc


def kernel(points, W_rel, b_rel, W_dist, b_dist, W_dens, b_dens, W_out, b_out):
    mrel = W_rel @ W_out[:SUB]
    vdist = W_dist @ W_out[SUB:2 * SUB]
    vdens = W_dens @ W_out[2 * SUB:]
    cvec = (b_rel @ W_out[:SUB] + b_dist @ W_out[SUB:2 * SUB]
            + b_dens @ W_out[2 * SUB:] + b_out)[None, :]
    pts_t = jnp.transpose(points, (0, 2, 1))

    return pl.pallas_call(
        _body,
        grid=(B, T),
        in_specs=[
            pl.BlockSpec((1, N, 3), lambda b, t: (b, 0, 0)),
            pl.BlockSpec((1, 3, N), lambda b, t: (b, 0, 0)),
            pl.BlockSpec((3, EMBED_DIM), lambda b, t: (0, 0)),
            pl.BlockSpec((1, EMBED_DIM), lambda b, t: (0, 0)),
            pl.BlockSpec((1, EMBED_DIM), lambda b, t: (0, 0)),
            pl.BlockSpec((1, EMBED_DIM), lambda b, t: (0, 0)),
        ],
        out_specs=pl.BlockSpec((1, ROWS, EMBED_DIM), lambda b, t: (b, t, 0)),
        out_shape=jax.ShapeDtypeStruct((B, N, EMBED_DIM), jnp.float32),
    )(points, pts_t, mrel, vdist, vdens, cvec)
